# baseline (speedup 1.0000x reference)
import jax
import jax.numpy as jnp
from jax.experimental import pallas as pl
from jax.experimental.pallas import tpu as pltpu

_D = 768
_H = _D // 2
_EPS = 1e-8
_THRESH = 0.5
_T = 1024


_NSUB = 4
_TS = _T // _NSUB


def _chunker_kernel(x_ref, wh_ref, bcat_ref, w2_ref, b2_ref,
                    f_ref, m_ref, klast_ref):
    j = pl.program_id(1)
    x = x_ref[0]
    w = wh_ref[...]
    bcat = bcat_ref[...]
    ys = []
    for i in range(_NSUB):
        xs = x[i * _TS:(i + 1) * _TS]
        ys.append(jnp.dot(xs.astype(jnp.bfloat16), w,
                          preferred_element_type=jnp.float32) + bcat)

    k_prev = klast_ref[...]
    for i, y in enumerate(ys):
        q = y[:, :_D]
        k = y[:, _D:2 * _D]
        h = jnp.maximum(y[:, 2 * _D:], 0.0)

        hb = h.astype(jnp.bfloat16).astype(jnp.float32)
        wb = w2_ref[...].astype(jnp.bfloat16).astype(jnp.float32)
        z = jnp.sum(hb * wb, axis=1, keepdims=True) + b2_ref[...]
        r = jax.nn.sigmoid(z)

        nq = jnp.sqrt(jnp.sum(q * q, axis=1, keepdims=True))
        nk = jnp.sqrt(jnp.sum(k * k, axis=1, keepdims=True))

        kshift = pltpu.roll(k, 1, 0)
        row_d = jax.lax.broadcasted_iota(jnp.int32, kshift.shape, 0)
        kshift = jnp.where(row_d == 0,
                           jnp.broadcast_to(k_prev, kshift.shape), kshift)
        nk0 = jnp.sqrt(jnp.sum(k_prev * k_prev, axis=1, keepdims=True))
        row_1 = jax.lax.broadcasted_iota(jnp.int32, nq.shape, 0)
        nkshift = jnp.where(row_1 == 0, nk0, pltpu.roll(nk, 1, 0))

        dots = jnp.sum(q * kshift, axis=1, keepdims=True)
        sim = dots / (jnp.maximum(nq, _EPS) * jnp.maximum(nkshift, _EPS))
        bp = 0.5 * (1.0 - sim)
        if i == 0:
            bp = jnp.where(jnp.logical_and(j == 0, row_1 == 0), 1.0, bp)

        final = 0.7 * bp + 0.3 * r
        sl = pl.ds(i * _TS, _TS)
        f_ref[0, sl] = final
        m_ref[0, sl] = (final > _THRESH).astype(jnp.int32)
        k_prev = k[-1:, :]

    klast_ref[...] = k_prev


def kernel(hidden_states, Wq, bq, Wk, bk, W1, b1, W2, b2):
    B, S, D = hidden_states.shape
    wcat = jnp.concatenate([Wq, Wk, W1], axis=1)
    wh = wcat.astype(jnp.bfloat16)
    bcat = jnp.concatenate([bq, bk, b1])[None, :]
    w2 = W2.reshape(1, _H)
    b2r = b2.reshape(1, 1)
    n_tiles = S // _T

    f, m = pl.pallas_call(
        _chunker_kernel,
        grid=(B, n_tiles),
        in_specs=[
            pl.BlockSpec((1, _T, D), lambda b, j: (b, j, 0)),
            pl.BlockSpec((D, 2 * _D + _H), lambda b, j: (0, 0)),
            pl.BlockSpec((1, 2 * _D + _H), lambda b, j: (0, 0)),
            pl.BlockSpec((1, _H), lambda b, j: (0, 0)),
            pl.BlockSpec((1, 1), lambda b, j: (0, 0)),
        ],
        out_specs=[
            pl.BlockSpec((1, _T, 1), lambda b, j: (b, j, 0)),
            pl.BlockSpec((1, _T, 1), lambda b, j: (b, j, 0)),
        ],
        out_shape=[
            jax.ShapeDtypeStruct((B, S, 1), jnp.float32),
            jax.ShapeDtypeStruct((B, S, 1), jnp.int32),
        ],
        scratch_shapes=[pltpu.VMEM((1, D), jnp.float32)],
        compiler_params=pltpu.CompilerParams(
            dimension_semantics=("arbitrary", "arbitrary"),
        ),
    )(hidden_states, wh, bcat, w2, b2r)
    return f[..., 0], m[..., 0]


# speedup vs baseline: 1.0391x; 1.0391x over previous
import jax
import jax.numpy as jnp
from jax.experimental import pallas as pl
from jax.experimental.pallas import tpu as pltpu

_D = 768
_H = _D // 2
_EPS = 1e-8
_THRESH = 0.5
_T = 2048


_NSUB = 4
_TS = _T // _NSUB


def _chunker_kernel(x_ref, wh_ref, bcat_ref, w2_ref, b2_ref,
                    f_ref, m_ref, klast_ref):
    j = pl.program_id(1)
    x = x_ref[0]
    w = wh_ref[...]
    bcat = bcat_ref[...]
    ys = []
    for i in range(_NSUB):
        xs = x[i * _TS:(i + 1) * _TS]
        ys.append(jnp.dot(xs.astype(jnp.bfloat16), w,
                          preferred_element_type=jnp.float32) + bcat)

    k_prev = klast_ref[...]
    for i, y in enumerate(ys):
        q = y[:, :_D]
        k = y[:, _D:2 * _D]
        h = jnp.maximum(y[:, 2 * _D:], 0.0)

        hb = h.astype(jnp.bfloat16).astype(jnp.float32)
        wb = w2_ref[...].astype(jnp.bfloat16).astype(jnp.float32)
        z = jnp.sum(hb * wb, axis=1, keepdims=True) + b2_ref[...]
        r = jax.nn.sigmoid(z)

        nq = jnp.sqrt(jnp.sum(q * q, axis=1, keepdims=True))
        nk = jnp.sqrt(jnp.sum(k * k, axis=1, keepdims=True))

        kshift = pltpu.roll(k, 1, 0)
        row_d = jax.lax.broadcasted_iota(jnp.int32, kshift.shape, 0)
        kshift = jnp.where(row_d == 0,
                           jnp.broadcast_to(k_prev, kshift.shape), kshift)
        nk0 = jnp.sqrt(jnp.sum(k_prev * k_prev, axis=1, keepdims=True))
        row_1 = jax.lax.broadcasted_iota(jnp.int32, nq.shape, 0)
        nkshift = jnp.where(row_1 == 0, nk0, pltpu.roll(nk, 1, 0))

        dots = jnp.sum(q * kshift, axis=1, keepdims=True)
        sim = dots / (jnp.maximum(nq, _EPS) * jnp.maximum(nkshift, _EPS))
        bp = 0.5 * (1.0 - sim)
        if i == 0:
            bp = jnp.where(jnp.logical_and(j == 0, row_1 == 0), 1.0, bp)

        final = 0.7 * bp + 0.3 * r
        sl = pl.ds(i * _TS, _TS)
        f_ref[0, sl] = final
        m_ref[0, sl] = (final > _THRESH).astype(jnp.int32)
        k_prev = k[-1:, :]

    klast_ref[...] = k_prev


def kernel(hidden_states, Wq, bq, Wk, bk, W1, b1, W2, b2):
    B, S, D = hidden_states.shape
    wcat = jnp.concatenate([Wq, Wk, W1], axis=1)
    wh = wcat.astype(jnp.bfloat16)
    bcat = jnp.concatenate([bq, bk, b1])[None, :]
    w2 = W2.reshape(1, _H)
    b2r = b2.reshape(1, 1)
    n_tiles = S // _T

    f, m = pl.pallas_call(
        _chunker_kernel,
        grid=(B, n_tiles),
        in_specs=[
            pl.BlockSpec((1, _T, D), lambda b, j: (b, j, 0)),
            pl.BlockSpec((D, 2 * _D + _H), lambda b, j: (0, 0)),
            pl.BlockSpec((1, 2 * _D + _H), lambda b, j: (0, 0)),
            pl.BlockSpec((1, _H), lambda b, j: (0, 0)),
            pl.BlockSpec((1, 1), lambda b, j: (0, 0)),
        ],
        out_specs=[
            pl.BlockSpec((1, _T, 1), lambda b, j: (b, j, 0)),
            pl.BlockSpec((1, _T, 1), lambda b, j: (b, j, 0)),
        ],
        out_shape=[
            jax.ShapeDtypeStruct((B, S, 1), jnp.float32),
            jax.ShapeDtypeStruct((B, S, 1), jnp.int32),
        ],
        scratch_shapes=[pltpu.VMEM((1, D), jnp.float32)],
        compiler_params=pltpu.CompilerParams(
            dimension_semantics=("arbitrary", "arbitrary"),
        ),
    )(hidden_states, wh, bcat, w2, b2r)
    return f[..., 0], m[..., 0]


# speedup vs baseline: 1.0427x; 1.0034x over previous
import jax
import jax.numpy as jnp
from jax.experimental import pallas as pl
from jax.experimental.pallas import tpu as pltpu

_D = 768
_H = _D // 2
_EPS = 1e-8
_THRESH = 0.5
_T = 4096


_NSUB = 8
_TS = _T // _NSUB


def _chunker_kernel(x_ref, wh_ref, bcat_ref, w2_ref, b2_ref,
                    f_ref, m_ref, klast_ref):
    j = pl.program_id(1)
    x = x_ref[0]
    w = wh_ref[...]
    bcat = bcat_ref[...]
    def sub_gemm(i):
        xs = x[i * _TS:(i + 1) * _TS]
        return jnp.dot(xs.astype(jnp.bfloat16), w,
                       preferred_element_type=jnp.float32) + bcat

    y_next = sub_gemm(0)
    k_prev = klast_ref[...]
    for i in range(_NSUB):
        y = y_next
        if i + 1 < _NSUB:
            y_next = sub_gemm(i + 1)
        q = y[:, :_D]
        k = y[:, _D:2 * _D]
        h = jnp.maximum(y[:, 2 * _D:], 0.0)

        hb = h.astype(jnp.bfloat16).astype(jnp.float32)
        wb = w2_ref[...].astype(jnp.bfloat16).astype(jnp.float32)
        z = jnp.sum(hb * wb, axis=1, keepdims=True) + b2_ref[...]
        r = jax.nn.sigmoid(z)

        nq = jnp.sqrt(jnp.sum(q * q, axis=1, keepdims=True))
        nk = jnp.sqrt(jnp.sum(k * k, axis=1, keepdims=True))

        kshift = pltpu.roll(k, 1, 0)
        row_d = jax.lax.broadcasted_iota(jnp.int32, kshift.shape, 0)
        kshift = jnp.where(row_d == 0,
                           jnp.broadcast_to(k_prev, kshift.shape), kshift)
        nk0 = jnp.sqrt(jnp.sum(k_prev * k_prev, axis=1, keepdims=True))
        row_1 = jax.lax.broadcasted_iota(jnp.int32, nq.shape, 0)
        nkshift = jnp.where(row_1 == 0, nk0, pltpu.roll(nk, 1, 0))

        dots = jnp.sum(q * kshift, axis=1, keepdims=True)
        sim = dots / (jnp.maximum(nq, _EPS) * jnp.maximum(nkshift, _EPS))
        bp = 0.5 * (1.0 - sim)
        if i == 0:
            bp = jnp.where(jnp.logical_and(j == 0, row_1 == 0), 1.0, bp)

        final = 0.7 * bp + 0.3 * r
        sl = pl.ds(i * _TS, _TS)
        f_ref[0, sl] = final
        m_ref[0, sl] = (final > _THRESH).astype(jnp.int32)
        k_prev = k[-1:, :]

    klast_ref[...] = k_prev


def kernel(hidden_states, Wq, bq, Wk, bk, W1, b1, W2, b2):
    B, S, D = hidden_states.shape
    wcat = jnp.concatenate([Wq, Wk, W1], axis=1)
    wh = wcat.astype(jnp.bfloat16)
    bcat = jnp.concatenate([bq, bk, b1])[None, :]
    w2 = W2.reshape(1, _H)
    b2r = b2.reshape(1, 1)
    n_tiles = S // _T

    f, m = pl.pallas_call(
        _chunker_kernel,
        grid=(B, n_tiles),
        in_specs=[
            pl.BlockSpec((1, _T, D), lambda b, j: (b, j, 0)),
            pl.BlockSpec((D, 2 * _D + _H), lambda b, j: (0, 0)),
            pl.BlockSpec((1, 2 * _D + _H), lambda b, j: (0, 0)),
            pl.BlockSpec((1, _H), lambda b, j: (0, 0)),
            pl.BlockSpec((1, 1), lambda b, j: (0, 0)),
        ],
        out_specs=[
            pl.BlockSpec((1, _T, 1), lambda b, j: (b, j, 0)),
            pl.BlockSpec((1, _T, 1), lambda b, j: (b, j, 0)),
        ],
        out_shape=[
            jax.ShapeDtypeStruct((B, S, 1), jnp.float32),
            jax.ShapeDtypeStruct((B, S, 1), jnp.int32),
        ],
        scratch_shapes=[pltpu.VMEM((1, D), jnp.float32)],
        compiler_params=pltpu.CompilerParams(
            dimension_semantics=("arbitrary", "arbitrary"),
        ),
    )(hidden_states, wh, bcat, w2, b2r)
    return f[..., 0], m[..., 0]


# speedup vs baseline: 1.0539x; 1.0108x over previous
import jax
import jax.numpy as jnp
from jax.experimental import pallas as pl
from jax.experimental.pallas import tpu as pltpu

_D = 768
_H = _D // 2
_EPS = 1e-8
_THRESH = 0.5
_T = 4096


_NSUB = 8
_TS = _T // _NSUB


def _chunker_kernel(x_ref, wh_ref, bcat_ref, w2_ref, b2_ref,
                    f_ref, m_ref, klast_ref):
    j = pl.program_id(1)
    x = x_ref[0]
    w = wh_ref[...]
    bcat = bcat_ref[...]
    def sub_gemm(i):
        xs = x[i * _TS:(i + 1) * _TS]
        return jnp.dot(xs.astype(jnp.bfloat16), w,
                       preferred_element_type=jnp.float32) + bcat

    y_next = sub_gemm(0)
    k_prev = klast_ref[...]
    for i in range(_NSUB):
        y = y_next
        if i + 1 < _NSUB:
            y_next = sub_gemm(i + 1)
        q = y[:, :_D]
        k = y[:, _D:2 * _D]
        h = jnp.maximum(y[:, 2 * _D:], 0.0)

        hb = h.astype(jnp.bfloat16).astype(jnp.float32)
        wb = w2_ref[...].astype(jnp.bfloat16).astype(jnp.float32)
        z = jnp.sum(hb * wb, axis=1, keepdims=True) + b2_ref[...]
        r = jax.nn.sigmoid(z)

        nq = jnp.sqrt(jnp.sum(q * q, axis=1, keepdims=True))
        nk = jnp.sqrt(jnp.sum(k * k, axis=1, keepdims=True))

        kshift = pltpu.roll(k, 1, 0)
        row_d = jax.lax.broadcasted_iota(jnp.int32, kshift.shape, 0)
        kshift = jnp.where(row_d == 0,
                           jnp.broadcast_to(k_prev, kshift.shape), kshift)
        nk0 = jnp.sqrt(jnp.sum(k_prev * k_prev, axis=1, keepdims=True))
        row_1 = jax.lax.broadcasted_iota(jnp.int32, nq.shape, 0)
        nkshift = jnp.where(row_1 == 0, nk0, pltpu.roll(nk, 1, 0))

        dots = jnp.sum(q * kshift, axis=1, keepdims=True)
        sim = dots / (jnp.maximum(nq, _EPS) * jnp.maximum(nkshift, _EPS))
        bp = 0.5 * (1.0 - sim)
        if i == 0:
            bp = jnp.where(jnp.logical_and(j == 0, row_1 == 0), 1.0, bp)

        final = 0.7 * bp + 0.3 * r
        sl = pl.ds(i * _TS, _TS)
        f_ref[0, sl] = final
        m_ref[0, sl] = (final > _THRESH).astype(jnp.int32)
        k_prev = k[-1:, :]

    klast_ref[...] = k_prev


def kernel(hidden_states, Wq, bq, Wk, bk, W1, b1, W2, b2):
    B, S, D = hidden_states.shape
    wcat = jnp.concatenate([Wq, Wk, W1], axis=1)
    wh = wcat.astype(jnp.bfloat16)
    bcat = jnp.concatenate([bq, bk, b1])[None, :]
    w2 = W2.reshape(1, _H)
    b2r = b2.reshape(1, 1)
    n_tiles = S // _T

    f, m = pl.pallas_call(
        _chunker_kernel,
        grid=(B, n_tiles),
        in_specs=[
            pl.BlockSpec((1, _T, D), lambda b, j: (b, j, 0)),
            pl.BlockSpec((D, 2 * _D + _H), lambda b, j: (0, 0)),
            pl.BlockSpec((1, 2 * _D + _H), lambda b, j: (0, 0)),
            pl.BlockSpec((1, _H), lambda b, j: (0, 0)),
            pl.BlockSpec((1, 1), lambda b, j: (0, 0)),
        ],
        out_specs=[
            pl.BlockSpec((1, _T, 1), lambda b, j: (b, j, 0)),
            pl.BlockSpec((1, _T, 1), lambda b, j: (b, j, 0)),
        ],
        out_shape=[
            jax.ShapeDtypeStruct((B, S, 1), jnp.float32),
            jax.ShapeDtypeStruct((B, S, 1), jnp.int32),
        ],
        scratch_shapes=[pltpu.VMEM((1, D), jnp.float32)],
        compiler_params=pltpu.CompilerParams(
            dimension_semantics=("parallel", "arbitrary"),
        ),
    )(hidden_states, wh, bcat, w2, b2r)
    return f[..., 0], m[..., 0]
